# Initial kernel scaffold; baseline (speedup 1.0000x reference)
import jax
import jax.numpy as jnp
from jax.experimental import pallas as pl


def kernel(vid0, vid1, flows):
    raise NotImplementedError("write your pallas kernel here")


# speedup vs baseline: 12.0567x; 12.0567x over previous
import jax
import jax.numpy as jnp
from jax.experimental import pallas as pl
from jax.experimental.pallas import tpu as pltpu

_T = 2
_C = 64
_H = 128
_W = 128
_PS = 7
_PAD = 3
_WS = 8
_KT = 7
_S0A = 8
_S0 = 4
_NG = _H // _S0
_NC = _H // _S0A
_NSH = 9
_YPAD = _H + 2 * _PAD
_YEXT = _YPAD + 2 * (_NSH // 2)

_HB = [abs(d - 4) + 4 for d in range(_WS)]


def _one_hot_f32(rows, cols, stride):
    r = jax.lax.broadcasted_iota(jnp.int32, (rows, cols), 0)
    c = jax.lax.broadcasted_iota(jnp.int32, (rows, cols), 1)
    return (c == stride * r).astype(jnp.float32)


def _corr_kernel(a_ref, b_ref, cf_ref, cc_ref):
    shi = pl.program_id(1)
    A = a_ref[0]
    Brows = b_ref[0, pl.ds(shi, _YPAD), :, :]
    sel4t = _one_hot_f32(_NG, _H, _S0)
    sel8t = _one_hot_f32(_NC, _H, _S0A)
    for swi in range(_NSH):
        Bs = Brows[:, swi:swi + _YPAD, :]
        E = jnp.sum(A * Bs, axis=-1)
        W1 = E[0:_H, :]
        for pi in range(1, _PS):
            W1 = W1 + E[pi:pi + _H, :]
        S1f = jnp.dot(sel4t, W1, preferred_element_type=jnp.float32, precision=jax.lax.Precision.HIGHEST)
        S1c = jnp.dot(sel8t, W1, preferred_element_type=jnp.float32, precision=jax.lax.Precision.HIGHEST)
        W2f = S1f[:, 0:_H]
        W2c = S1c[:, 0:_H]
        for pj in range(1, _PS):
            W2f = W2f + S1f[:, pj:pj + _H]
            W2c = W2c + S1c[:, pj:pj + _H]
        cf_ref[0, 0, swi] = jnp.dot(W2f, sel4t.T, preferred_element_type=jnp.float32, precision=jax.lax.Precision.HIGHEST)
        cc_ref[0, 0, swi] = jnp.dot(W2c, sel8t.T, preferred_element_type=jnp.float32, precision=jax.lax.Precision.HIGHEST)


def _select_kernel(cf_ref, cc_ref, df_ref, ft_ref, fh_ref, fw_ref):
    Cf = cf_ref[...]
    Cc = cc_ref[...]

    D_int = Cc[:, 0:_WS, 0:_WS]
    Chb = jnp.stack([Cc[:, j] for j in _HB], axis=1)
    Cwb = jnp.stack([Cc[:, :, j] for j in _HB], axis=2)
    Chw = jnp.stack([Chb[:, :, j] for j in _HB], axis=2)
    gh_i = jax.lax.broadcasted_iota(jnp.int32, (4, _WS, _WS, _NC, _NC), 3)
    gw_i = jax.lax.broadcasted_iota(jnp.int32, (4, _WS, _WS, _NC, _NC), 4)
    row0 = gh_i == 0
    col0 = gw_i == 0
    rowv = jnp.broadcast_to(Chb[:, :, 0:_WS, 0:1, :], D_int.shape)
    colv = jnp.broadcast_to(Cwb[:, 0:_WS, :, :, 0:1], D_int.shape)
    corv = jnp.broadcast_to(Chw[:, :, :, 0:1, 0:1], D_int.shape)
    D = jnp.where(row0 & col0, corv,
                  jnp.where(row0, rowv, jnp.where(col0, colv, D_int)))
    dists = jnp.stack([D[0], D[0], D[1], D[2], D[3], D[3]], axis=0)
    dists = dists.reshape(_T, 3, _WS, _WS, _NC, _NC)

    dt_i = jax.lax.broadcasted_iota(jnp.int32, dists.shape, 1)
    dh_i = jax.lax.broadcasted_iota(jnp.int32, dists.shape, 2)
    dw_i = jax.lax.broadcasted_iota(jnp.int32, dists.shape, 3)
    kid = dt_i * 64 + dh_i * 8 + dw_i

    tq_i = jax.lax.broadcasted_iota(jnp.int32, (_T, _NC, _NC), 0)
    ch_i = jax.lax.broadcasted_iota(jnp.int32, (_T, _NC, _NC), 1)
    cw_i = jax.lax.broadcasted_iota(jnp.int32, (_T, _NC, _NC), 2)

    fi = jax.lax.broadcasted_iota(jnp.int32, (_NG, _NC), 0)
    gi = jax.lax.broadcasted_iota(jnp.int32, (_NG, _NC), 1)
    up = (fi // 2 == gi).astype(jnp.float32)
    upt = up.T

    if_i = jax.lax.broadcasted_iota(jnp.int32, (_T, _NG, _NG), 1)
    jf_i = jax.lax.broadcasted_iota(jnp.int32, (_T, _NG, _NG), 2)
    dhf = (if_i % 2) * _S0
    dwf = (jf_i % 2) * _S0
    g8h = (if_i // 2) * _S0A
    g8w = (jf_i // 2) * _S0A

    def upsample(x_f32):
        outs = []
        for t in range(_T):
            outs.append(jnp.dot(jnp.dot(up, x_f32[t], preferred_element_type=jnp.float32, precision=jax.lax.Precision.HIGHEST),
                                upt, preferred_element_type=jnp.float32, precision=jax.lax.Precision.HIGHEST))
        return jnp.stack(outs, axis=0)

    neg_inf = jnp.float32(-jnp.inf)
    for i in range(_KT):
        m = jnp.max(dists, axis=(1, 2, 3), keepdims=True)
        cand = jnp.min(jnp.where(dists == m, kid, 192), axis=(1, 2, 3),
                       keepdims=True)
        dists = jnp.where(kid == cand, neg_inf, dists)
        idx = cand[:, 0, 0, 0]
        dt = idx // 64
        dh = (idx - dt * 64) // 8
        dw = idx - dt * 64 - dh * 8
        it = jnp.clip(tq_i + dt - 1, 0, _T - 1)
        ih = jnp.abs(ch_i * _S0A + dh - 4)
        iw = jnp.abs(cw_i * _S0A + dw - 4)
        it_u = upsample(it.astype(jnp.float32))
        ih_u = upsample(ih.astype(jnp.float32))
        iw_u = upsample(iw.astype(jnp.float32))
        fh = ih_u + dhf.astype(jnp.float32)
        fw = iw_u + dwf.astype(jnp.float32)
        ft_ref[i] = it_u.astype(jnp.int32)
        fh_ref[i] = jnp.clip(fh, 0, _H - 1).astype(jnp.int32)
        fw_ref[i] = jnp.clip(fw, 0, _W - 1).astype(jnp.int32)
        it_i = it_u.astype(jnp.int32)
        rh_i = (ih_u.astype(jnp.int32) - g8h) + 4
        rw_i = (iw_u.astype(jnp.int32) - g8w) + 4
        for t in range(_T):
            pair = 2 * t + it_i[t]
            P1 = jnp.zeros((_NSH, _NSH, _NG, _NG), jnp.float32)
            for p in range(4):
                P1 = P1 + jnp.where(pair[None, None] == p, Cf[p], 0.0)
            P2 = jnp.zeros((_NSH, _NG, _NG), jnp.float32)
            for a in range(_NSH):
                P2 = P2 + jnp.where(rh_i[t][None] == a, P1[a], 0.0)
            acc = jnp.zeros((_NG, _NG), jnp.float32)
            for b in range(_NSH):
                acc = acc + jnp.where(rw_i[t] == b, P2[b], 0.0)
            df_ref[i, t] = acc


def kernel(vid0, vid1, flows):
    del flows
    B, T, C, H, W = vid0.shape
    pw = ((0, 0), (0, 0), (_PAD, _PAD), (_PAD, _PAD))
    p0 = jnp.pad(vid0[0], pw, mode="reflect")
    p1 = jnp.pad(vid1[0], pw, mode="reflect")
    ew = ((0, 0), (0, 0), (4, 4), (4, 4))
    p1e = jnp.pad(p1, ew)
    p0t = p0.transpose(0, 2, 3, 1)
    p1t = p1e.transpose(0, 2, 3, 1)

    cf, cc = pl.pallas_call(
        _corr_kernel,
        grid=(4, _NSH),
        in_specs=[
            pl.BlockSpec((1, _YPAD, _YPAD, _C), lambda p, i: (p // 2, 0, 0, 0)),
            pl.BlockSpec((1, _YEXT, _YEXT, _C), lambda p, i: (p % 2, 0, 0, 0)),
        ],
        out_specs=[
            pl.BlockSpec((1, 1, _NSH, _NG, _NG), lambda p, i: (p, i, 0, 0, 0)),
            pl.BlockSpec((1, 1, _NSH, _NC, _NC), lambda p, i: (p, i, 0, 0, 0)),
        ],
        out_shape=[
            jax.ShapeDtypeStruct((4, _NSH, _NSH, _NG, _NG), jnp.float32),
            jax.ShapeDtypeStruct((4, _NSH, _NSH, _NC, _NC), jnp.float32),
        ],
    )(p0t, p1t)

    df, ft, fh, fw = pl.pallas_call(
        _select_kernel,
        out_shape=[
            jax.ShapeDtypeStruct((_KT, _T, _NG, _NG), jnp.float32),
            jax.ShapeDtypeStruct((_KT, _T, _NG, _NG), jnp.int32),
            jax.ShapeDtypeStruct((_KT, _T, _NG, _NG), jnp.int32),
            jax.ShapeDtypeStruct((_KT, _T, _NG, _NG), jnp.int32),
        ],
    )(cf, cc)

    qf = _T * _NG * _NG
    dists_f = df.transpose(1, 2, 3, 0).reshape(1, 1, qf, _KT)
    inds_f = jnp.stack([ft, fh, fw], axis=-1)
    inds_f = inds_f.transpose(1, 2, 3, 0, 4).reshape(1, 1, qf, _KT, 3)
    return dists_f, inds_f.astype(jnp.int32)
